# baseline (speedup 1.0000x reference)
import functools

import jax
import jax.numpy as jnp
from jax import lax
from jax.experimental import pallas as pl
from jax.experimental.pallas import tpu as pltpu
from jax.experimental.pallas import tpu_sc as plsc


def _mm_body(x_ref, w_ref, b_ref, o_ref):
    xt = x_ref[...]
    o_ref[...] = jnp.dot(xt.astype(jnp.bfloat16), w_ref[...],
                         preferred_element_type=jnp.float32) + b_ref[...]


def _matmul(xr, wb, br, N, D, TS):
    return pl.pallas_call(
        _mm_body,
        grid=(N // TS,),
        in_specs=[
            pl.BlockSpec((TS, D), lambda i: (i, 0)),
            pl.BlockSpec((D, D), lambda i: (0, 0)),
            pl.BlockSpec((1, D), lambda i: (0, 0)),
        ],
        out_specs=pl.BlockSpec((TS, D), lambda i: (i, 0)),
        out_shape=jax.ShapeDtypeStruct((N, D), jnp.float32),
        compiler_params=pltpu.CompilerParams(
            dimension_semantics=("arbitrary",),
        ),
    )(xr, wb, br)


def _sc_scatter_body(S, K, y_ref, acts_ref, mask_ref, idx_v, rows_v, sem):
    c = lax.axis_index("c")
    s = lax.axis_index("s")
    wid = s * 2 + c
    per = 16
    nparts = K // per

    @pl.when(wid < 2 * nparts)
    def _():
        batch = wid // nparts
        off = (wid % nparts) * per
        pltpu.sync_copy(mask_ref.at[pl.ds(off, per)], idx_v)
        idx = idx_v[...] + batch * S
        pltpu.sync_copy(acts_ref.at[pl.ds(off, per)], rows_v)
        pltpu.async_copy(rows_v, y_ref.at[idx], sem).wait()


def _sc_scatter(y2d, acts, mask_idxs, S, K, D):
    mesh = plsc.VectorSubcoreMesh(core_axis_name="c", subcore_axis_name="s")
    body = functools.partial(_sc_scatter_body, S, K)
    fn = pl.kernel(
        body,
        (),
        mesh=mesh,
        scratch_types=[
            pltpu.VMEM((16,), jnp.int32),
            pltpu.VMEM((16, D), jnp.float32),
            pltpu.SemaphoreType.DMA,
        ],
        compiler_params=pltpu.CompilerParams(has_side_effects=True),
    )
    fn(y2d, acts, mask_idxs)


def kernel(x, W, b, acts, mask_idxs):
    B, S, D = x.shape
    K = mask_idxs.shape[0]
    N = B * S
    xr = x.reshape(N, D)
    wb = W.astype(jnp.bfloat16)
    br = b.reshape(1, D)
    y = _matmul(xr, wb, br, N, D, 512)
    _sc_scatter(y, acts, mask_idxs, S, K, D)
    return y.reshape(B, S, D)
